# baseline (speedup 1.0000x reference)
import functools

import jax
import jax.numpy as jnp
from jax import lax
from jax.experimental import pallas as pl
from jax.experimental.pallas import tpu as pltpu
from jax.experimental.pallas import tpu_sc as plsc

_N = 10000
_DIM = 128
_HALF = 64
_E = 320000
_NSUB = 16
_CHUNK = 112
_IBLK = 8
_NBLK = 23
_CPT = _IBLK * _NBLK
_EPAD = _NSUB * _CPT * _CHUNK
_NROWS = 10016
_ZCP = _NROWS // _NSUB
_OPT = _N // _NSUB


def _sc_body(hs, av, srcr, dstr, uv, out_h, out_f,
             sg, dg, ds, rh0, ra0, rh1, ra1, rh2, ra2, ul,
             acc_h, acc_f, gh0, ga0, gh1, ga1, gh2, ga2,
             sh0, sf0, sh1, sf1, sh2, sf2):
    c = lax.axis_index("c")
    s = lax.axis_index("s")
    bias = c * _N

    pltpu.sync_copy(uv.at[pl.ds(c * _HALF, _HALF)], ul)

    def _zero(r, carry):
        for j in range(4):
            rh0[r, pl.ds(j * 16, 16)] = jnp.zeros((16,), jnp.float32)
        return carry

    lax.fori_loop(0, _CHUNK, _zero, 0)
    zbase = s * _ZCP
    for k in range(_ZCP // _CHUNK):
        pltpu.sync_copy(rh0, acc_h.at[pl.ds(zbase + k * _CHUNK, _CHUNK)])
        pltpu.sync_copy(rh0, acc_f.at[pl.ds(zbase + k * _CHUNK, _CHUNK)])
    _zrem = _ZCP % _CHUNK
    if _zrem:
        zoff = zbase + (_ZCP // _CHUNK) * _CHUNK
        pltpu.sync_copy(rh0.at[pl.ds(0, _zrem)], acc_h.at[pl.ds(zoff, _zrem)])
        pltpu.sync_copy(rh0.at[pl.ds(0, _zrem)], acc_f.at[pl.ds(zoff, _zrem)])
    plsc.subcore_barrier()

    uj = [ul[pl.ds(j * 16, 16)] for j in range(4)]
    one = jnp.ones((16,), jnp.float32)

    bufs = [(rh0, ra0, gh0, ga0, sh0, sf0),
            (rh1, ra1, gh1, ga1, sh1, sf1),
            (rh2, ra2, gh2, ga2, sh2, sf2)]

    def _compute(rh, ra):
        def _frow(r, rcarry):
            for j in range(4):
                sl = pl.ds(j * 16, 16)
                z = ra[r, sl] - uj[j] * rh[r, sl]
                ra[r, sl] = one / (one + jnp.exp(z))
            return rcarry

        lax.fori_loop(0, _CHUNK, _frow, 0)

    def _block(g, carry):
        pltpu.sync_copy(srcr.at[s, pl.ds(g * _IBLK, _IBLK)], sg)
        pltpu.sync_copy(dstr.at[s, pl.ds(g * _IBLK, _IBLK)], ds)
        for r in range(_IBLK):
            for j in range(_CHUNK // 16):
                sl = pl.ds(j * 16, 16)
                sg[r, sl] = sg[r, sl] + bias
                dg[r, sl] = jnp.minimum(ds[r, sl], _N - 1) + bias

        def _issue_g(i):
            rh, ra, gh, ga, _, _ = bufs[i % 3]
            dh = pltpu.async_copy(hs.at[sg.at[i]], rh, gh)
            da = pltpu.async_copy(av.at[dg.at[i]], ra, ga)
            return (dh, da)

        gd = [None] * _IBLK
        sd = [None] * _IBLK
        gd[0] = _issue_g(0)
        for i in range(_IBLK):
            rh, ra, _, _, sh, sf = bufs[i % 3]
            if i + 1 < _IBLK:
                if i - 2 >= 0:
                    sd[i - 2][0].wait()
                    sd[i - 2][1].wait()
                gd[i + 1] = _issue_g(i + 1)
            gd[i][0].wait()
            gd[i][1].wait()
            _compute(rh, ra)
            dh = pltpu.async_copy(rh, acc_h.at[ds.at[i]], sh, add=True)
            df = pltpu.async_copy(ra, acc_f.at[ds.at[i]], sf, add=True)
            sd[i] = (dh, df)
        for i in (_IBLK - 2, _IBLK - 1):
            sd[i][0].wait()
            sd[i][1].wait()
        return carry

    lax.fori_loop(0, _NBLK, _block, 0)
    plsc.subcore_barrier()

    obase = s * _OPT
    pltpu.sync_copy(acc_h.at[pl.ds(obase, _OPT)],
                    out_h.at[pl.ds(obase, _OPT), pl.ds(c * _HALF, _HALF)])
    pltpu.sync_copy(acc_f.at[pl.ds(obase, _OPT)],
                    out_f.at[pl.ds(obase, _OPT), pl.ds(c * _HALF, _HALF)])


_sc_seg = functools.partial(
    pl.kernel,
    out_type=[
        jax.ShapeDtypeStruct((_N, _DIM), jnp.float32),
        jax.ShapeDtypeStruct((_N, _DIM), jnp.float32),
    ],
    mesh=plsc.VectorSubcoreMesh(core_axis_name="c", subcore_axis_name="s"),
    scratch_types=[
        pltpu.VMEM((_IBLK, _CHUNK), jnp.int32),
        pltpu.VMEM((_IBLK, _CHUNK), jnp.int32),
        pltpu.VMEM((_IBLK, _CHUNK), jnp.int32),
        pltpu.VMEM((_CHUNK, _HALF), jnp.float32),
        pltpu.VMEM((_CHUNK, _HALF), jnp.float32),
        pltpu.VMEM((_CHUNK, _HALF), jnp.float32),
        pltpu.VMEM((_CHUNK, _HALF), jnp.float32),
        pltpu.VMEM((_CHUNK, _HALF), jnp.float32),
        pltpu.VMEM((_CHUNK, _HALF), jnp.float32),
        pltpu.VMEM((_HALF,), jnp.float32),
        pltpu.VMEM_SHARED((_NROWS, _HALF), jnp.float32),
        pltpu.VMEM_SHARED((_NROWS, _HALF), jnp.float32),
    ] + [pltpu.SemaphoreType.DMA] * 12,
    compiler_params=pltpu.CompilerParams(use_tc_tiling_on_sc=False),
)(_sc_body)


def _prep_body(x_ref, h_ref, w_ref, b_ref, hs_ref, as_ref):
    cidx = pl.program_id(0)
    xb = x_ref[...]
    hb = h_ref[...]
    ab = -(w_ref[...] * xb + b_ref[...])
    hs_ref[...] = jnp.where(cidx == 0, hb[:, 0, :], hb[:, 1, :])
    as_ref[...] = jnp.where(cidx == 0, ab[:, 0, :], ab[:, 1, :])


def _prep(x, h, w_for, b_for):
    blk = 1000
    full_in = pl.BlockSpec((blk, 2, _HALF), lambda c, i: (i, 0, 0))
    vec = pl.BlockSpec((1, 2, _HALF), lambda c, i: (0, 0, 0))
    out = pl.BlockSpec((blk, _HALF), lambda c, i: (c * (_N // blk) + i, 0))
    return pl.pallas_call(
        _prep_body,
        grid=(2, _N // blk),
        in_specs=[full_in, full_in, vec, vec],
        out_specs=[out, out],
        out_shape=[
            jax.ShapeDtypeStruct((2 * _N, _HALF), jnp.float32),
            jax.ShapeDtypeStruct((2 * _N, _HALF), jnp.float32),
        ],
    )(x.reshape(_N, 2, _HALF), h.reshape(_N, 2, _HALF),
      w_for.reshape(1, 2, _HALF), b_for.reshape(1, 2, _HALF))


def _gates_body(x_ref, hs_ref, fs_ref, wi, ui, bi, wc, uc, bc, wo, uo, bo,
                ht_ref, ct_ref):
    x = x_ref[...]
    hsum = hs_ref[...]
    fsum = fs_ref[...]
    it = jax.nn.sigmoid(wi[...] * x + ui[...] * hsum + bi[...])
    ctt = jnp.tanh(wc[...] * x + uc[...] * hsum + bc[...])
    ct = it * ctt + fsum
    ot = jax.nn.sigmoid(wo[...] * x + uo[...] * hsum + bo[...])
    ht_ref[...] = ot * jnp.tanh(ct)
    ct_ref[...] = ct


def _gates(x, hsum, fsum, wi, ui, bi, wc, uc, bc, wo, uo, bo):
    blk = 1000
    grid = _N // blk
    row = pl.BlockSpec((blk, _DIM), lambda i: (i, 0))
    vec = pl.BlockSpec((1, _DIM), lambda i: (0, 0))
    return pl.pallas_call(
        _gates_body,
        grid=(grid,),
        in_specs=[row, row, row] + [vec] * 9,
        out_specs=[row, row],
        out_shape=[
            jax.ShapeDtypeStruct((_N, _DIM), jnp.float32),
            jax.ShapeDtypeStruct((_N, _DIM), jnp.float32),
        ],
    )(x, hsum, fsum, wi, ui, bi, wc, uc, bc, wo, uo, bo)


def kernel(x, h, w_for, u_for, b_for, w_in, u_in, b_in, w_ce, u_ce, b_ce,
           w_out, u_out, b_out, edge_index):
    src = edge_index[0].astype(jnp.int32)
    dst = edge_index[1].astype(jnp.int32)
    pad = _EPAD - _E
    srcr = jnp.concatenate([src, jnp.zeros((pad,), jnp.int32)])
    dstr = jnp.concatenate([dst, jnp.full((pad,), _N, jnp.int32)])
    srcr = srcr.reshape(_NSUB, _CPT, _CHUNK)
    dstr = dstr.reshape(_NSUB, _CPT, _CHUNK)
    hs, av = _prep(x, h, w_for, b_for)

    chs, cfs = _sc_seg(hs, av, srcr, dstr, u_for)

    r = lambda v: v.reshape(1, _DIM)
    ht, ct = _gates(x, chs, cfs, r(w_in), r(u_in), r(b_in), r(w_ce), r(u_ce),
                    r(b_ce), r(w_out), r(u_out), r(b_out))
    return ht, ct


# speedup vs baseline: 1.1898x; 1.1898x over previous
import functools

import jax
import jax.numpy as jnp
from jax import lax
from jax.experimental import pallas as pl
from jax.experimental.pallas import tpu as pltpu
from jax.experimental.pallas import tpu_sc as plsc

_N = 10000
_DIM = 128
_HALF = 64
_E = 320000
_NSUB = 16
_CHUNK = 112
_IBLK = 8
_NBLK = 23
_CPT = _IBLK * _NBLK
_EPAD = _NSUB * _CPT * _CHUNK
_NROWS = 10016
_ZCP = _NROWS // _NSUB
_OPT = _N // _NSUB


def _sc_body(hs, av, srcr, dstr, uv, out_h, out_f,
             sg, dg, ds, rh0, ra0, rh1, ra1, rh2, ra2, ul,
             acc_h, acc_f, gh0, ga0, gh1, ga1, gh2, ga2,
             sh0, sf0, sh1, sf1, sh2, sf2):
    c = lax.axis_index("c")
    s = lax.axis_index("s")
    bias = c * _N

    pltpu.sync_copy(uv.at[pl.ds(c * _HALF, _HALF)], ul)

    def _zero(r, carry):
        for j in range(4):
            rh0[r, pl.ds(j * 16, 16)] = jnp.zeros((16,), jnp.float32)
        return carry

    lax.fori_loop(0, _CHUNK, _zero, 0)
    zbase = s * _ZCP
    for k in range(_ZCP // _CHUNK):
        pltpu.sync_copy(rh0, acc_h.at[pl.ds(zbase + k * _CHUNK, _CHUNK)])
        pltpu.sync_copy(rh0, acc_f.at[pl.ds(zbase + k * _CHUNK, _CHUNK)])
    _zrem = _ZCP % _CHUNK
    if _zrem:
        zoff = zbase + (_ZCP // _CHUNK) * _CHUNK
        pltpu.sync_copy(rh0.at[pl.ds(0, _zrem)], acc_h.at[pl.ds(zoff, _zrem)])
        pltpu.sync_copy(rh0.at[pl.ds(0, _zrem)], acc_f.at[pl.ds(zoff, _zrem)])
    plsc.subcore_barrier()

    uj = [ul[pl.ds(j * 16, 16)] for j in range(4)]
    one = jnp.ones((16,), jnp.float32)

    bufs = [(rh0, ra0, gh0, ga0, sh0, sf0),
            (rh1, ra1, gh1, ga1, sh1, sf1),
            (rh2, ra2, gh2, ga2, sh2, sf2)]

    def _compute(rh, ra):
        def _frow(q, rcarry):
            r0 = q * 4
            for r in range(4):
                for j in range(4):
                    sl = pl.ds(j * 16, 16)
                    z = ra[r0 + r, sl] - uj[j] * rh[r0 + r, sl]
                    ra[r0 + r, sl] = one / (one + jnp.exp(z))
            return rcarry

        lax.fori_loop(0, _CHUNK // 4, _frow, 0)

    def _block(g, carry):
        pltpu.sync_copy(srcr.at[s, pl.ds(g * _IBLK, _IBLK)], sg)
        pltpu.sync_copy(dstr.at[s, pl.ds(g * _IBLK, _IBLK)], ds)
        for r in range(_IBLK):
            for j in range(_CHUNK // 16):
                sl = pl.ds(j * 16, 16)
                sg[r, sl] = sg[r, sl] + bias
                dg[r, sl] = jnp.minimum(ds[r, sl], _N - 1) + bias

        def _issue_g(i):
            rh, ra, gh, ga, _, _ = bufs[i % 3]
            dh = pltpu.async_copy(hs.at[sg.at[i]], rh, gh)
            da = pltpu.async_copy(av.at[dg.at[i]], ra, ga)
            return (dh, da)

        gd = [None] * _IBLK
        sd = [None] * _IBLK
        gd[0] = _issue_g(0)
        for i in range(_IBLK):
            rh, ra, _, _, sh, sf = bufs[i % 3]
            if i + 1 < _IBLK:
                if i - 2 >= 0:
                    sd[i - 2][0].wait()
                    sd[i - 2][1].wait()
                gd[i + 1] = _issue_g(i + 1)
            gd[i][0].wait()
            gd[i][1].wait()
            _compute(rh, ra)
            dh = pltpu.async_copy(rh, acc_h.at[ds.at[i]], sh, add=True)
            df = pltpu.async_copy(ra, acc_f.at[ds.at[i]], sf, add=True)
            sd[i] = (dh, df)
        for i in (_IBLK - 2, _IBLK - 1):
            sd[i][0].wait()
            sd[i][1].wait()
        return carry

    lax.fori_loop(0, _NBLK, _block, 0)
    plsc.subcore_barrier()

    obase = s * _OPT
    pltpu.sync_copy(acc_h.at[pl.ds(obase, _OPT)],
                    out_h.at[pl.ds(obase, _OPT), pl.ds(c * _HALF, _HALF)])
    pltpu.sync_copy(acc_f.at[pl.ds(obase, _OPT)],
                    out_f.at[pl.ds(obase, _OPT), pl.ds(c * _HALF, _HALF)])


_sc_seg = functools.partial(
    pl.kernel,
    out_type=[
        jax.ShapeDtypeStruct((_N, _DIM), jnp.float32),
        jax.ShapeDtypeStruct((_N, _DIM), jnp.float32),
    ],
    mesh=plsc.VectorSubcoreMesh(core_axis_name="c", subcore_axis_name="s"),
    scratch_types=[
        pltpu.VMEM((_IBLK, _CHUNK), jnp.int32),
        pltpu.VMEM((_IBLK, _CHUNK), jnp.int32),
        pltpu.VMEM((_IBLK, _CHUNK), jnp.int32),
        pltpu.VMEM((_CHUNK, _HALF), jnp.float32),
        pltpu.VMEM((_CHUNK, _HALF), jnp.float32),
        pltpu.VMEM((_CHUNK, _HALF), jnp.float32),
        pltpu.VMEM((_CHUNK, _HALF), jnp.float32),
        pltpu.VMEM((_CHUNK, _HALF), jnp.float32),
        pltpu.VMEM((_CHUNK, _HALF), jnp.float32),
        pltpu.VMEM((_HALF,), jnp.float32),
        pltpu.VMEM_SHARED((_NROWS, _HALF), jnp.float32),
        pltpu.VMEM_SHARED((_NROWS, _HALF), jnp.float32),
    ] + [pltpu.SemaphoreType.DMA] * 12,
    compiler_params=pltpu.CompilerParams(use_tc_tiling_on_sc=False),
)(_sc_body)


def _prep_body(x_ref, h_ref, w_ref, b_ref, hs_ref, as_ref):
    cidx = pl.program_id(0)
    xb = x_ref[...]
    hb = h_ref[...]
    ab = -(w_ref[...] * xb + b_ref[...])
    hs_ref[...] = jnp.where(cidx == 0, hb[:, 0, :], hb[:, 1, :])
    as_ref[...] = jnp.where(cidx == 0, ab[:, 0, :], ab[:, 1, :])


def _prep(x, h, w_for, b_for):
    blk = 1000
    full_in = pl.BlockSpec((blk, 2, _HALF), lambda c, i: (i, 0, 0))
    vec = pl.BlockSpec((1, 2, _HALF), lambda c, i: (0, 0, 0))
    out = pl.BlockSpec((blk, _HALF), lambda c, i: (c * (_N // blk) + i, 0))
    return pl.pallas_call(
        _prep_body,
        grid=(2, _N // blk),
        in_specs=[full_in, full_in, vec, vec],
        out_specs=[out, out],
        out_shape=[
            jax.ShapeDtypeStruct((2 * _N, _HALF), jnp.float32),
            jax.ShapeDtypeStruct((2 * _N, _HALF), jnp.float32),
        ],
    )(x.reshape(_N, 2, _HALF), h.reshape(_N, 2, _HALF),
      w_for.reshape(1, 2, _HALF), b_for.reshape(1, 2, _HALF))


def _gates_body(x_ref, hs_ref, fs_ref, wi, ui, bi, wc, uc, bc, wo, uo, bo,
                ht_ref, ct_ref):
    x = x_ref[...]
    hsum = hs_ref[...]
    fsum = fs_ref[...]
    it = jax.nn.sigmoid(wi[...] * x + ui[...] * hsum + bi[...])
    ctt = jnp.tanh(wc[...] * x + uc[...] * hsum + bc[...])
    ct = it * ctt + fsum
    ot = jax.nn.sigmoid(wo[...] * x + uo[...] * hsum + bo[...])
    ht_ref[...] = ot * jnp.tanh(ct)
    ct_ref[...] = ct


def _gates(x, hsum, fsum, wi, ui, bi, wc, uc, bc, wo, uo, bo):
    blk = 1000
    grid = _N // blk
    row = pl.BlockSpec((blk, _DIM), lambda i: (i, 0))
    vec = pl.BlockSpec((1, _DIM), lambda i: (0, 0))
    return pl.pallas_call(
        _gates_body,
        grid=(grid,),
        in_specs=[row, row, row] + [vec] * 9,
        out_specs=[row, row],
        out_shape=[
            jax.ShapeDtypeStruct((_N, _DIM), jnp.float32),
            jax.ShapeDtypeStruct((_N, _DIM), jnp.float32),
        ],
    )(x, hsum, fsum, wi, ui, bi, wc, uc, bc, wo, uo, bo)


def kernel(x, h, w_for, u_for, b_for, w_in, u_in, b_in, w_ce, u_ce, b_ce,
           w_out, u_out, b_out, edge_index):
    src = edge_index[0].astype(jnp.int32)
    dst = edge_index[1].astype(jnp.int32)
    pad = _EPAD - _E
    srcr = jnp.concatenate([src, jnp.zeros((pad,), jnp.int32)])
    dstr = jnp.concatenate([dst, jnp.full((pad,), _N, jnp.int32)])
    srcr = srcr.reshape(_NSUB, _CPT, _CHUNK)
    dstr = dstr.reshape(_NSUB, _CPT, _CHUNK)
    hs, av = _prep(x, h, w_for, b_for)

    chs, cfs = _sc_seg(hs, av, srcr, dstr, u_for)

    r = lambda v: v.reshape(1, _DIM)
    ht, ct = _gates(x, chs, cfs, r(w_in), r(u_in), r(b_in), r(w_ce), r(u_ce),
                    r(b_ce), r(w_out), r(u_out), r(b_out))
    return ht, ct


# speedup vs baseline: 1.2769x; 1.0732x over previous
import functools

import jax
import jax.numpy as jnp
from jax import lax
from jax.experimental import pallas as pl
from jax.experimental.pallas import tpu as pltpu
from jax.experimental.pallas import tpu_sc as plsc

_N = 10000
_DIM = 128
_HALF = 64
_E = 320000
_NSUB = 16
_CHUNK = 112
_IBLK = 23
_NBLK = 8
_CPT = _IBLK * _NBLK
_EPAD = _NSUB * _CPT * _CHUNK
_NROWS = 10016
_ZCP = _NROWS // _NSUB
_OPT = _N // _NSUB


def _sc_body(hs, av, srcr, dstr, uv, out_h, out_f,
             sg, dg, ds, rh0, ra0, rh1, ra1, rh2, ra2, ul,
             acc_h, acc_f, gh0, ga0, gh1, ga1, gh2, ga2,
             sh0, sf0, sh1, sf1, sh2, sf2):
    c = lax.axis_index("c")
    s = lax.axis_index("s")
    bias = c * _N

    pltpu.sync_copy(uv.at[pl.ds(c * _HALF, _HALF)], ul)

    def _zero(r, carry):
        for j in range(4):
            rh0[r, pl.ds(j * 16, 16)] = jnp.zeros((16,), jnp.float32)
        return carry

    lax.fori_loop(0, _CHUNK, _zero, 0)
    zbase = s * _ZCP
    for k in range(_ZCP // _CHUNK):
        pltpu.sync_copy(rh0, acc_h.at[pl.ds(zbase + k * _CHUNK, _CHUNK)])
        pltpu.sync_copy(rh0, acc_f.at[pl.ds(zbase + k * _CHUNK, _CHUNK)])
    _zrem = _ZCP % _CHUNK
    if _zrem:
        zoff = zbase + (_ZCP // _CHUNK) * _CHUNK
        pltpu.sync_copy(rh0.at[pl.ds(0, _zrem)], acc_h.at[pl.ds(zoff, _zrem)])
        pltpu.sync_copy(rh0.at[pl.ds(0, _zrem)], acc_f.at[pl.ds(zoff, _zrem)])
    plsc.subcore_barrier()

    uj = [ul[pl.ds(j * 16, 16)] for j in range(4)]
    one = jnp.ones((16,), jnp.float32)

    bufs = [(rh0, ra0, gh0, ga0, sh0, sf0),
            (rh1, ra1, gh1, ga1, sh1, sf1),
            (rh2, ra2, gh2, ga2, sh2, sf2)]

    def _compute(rh, ra):
        def _frow(q, rcarry):
            r0 = q * 8
            for r in range(8):
                for j in range(4):
                    sl = pl.ds(j * 16, 16)
                    z = ra[r0 + r, sl] - uj[j] * rh[r0 + r, sl]
                    ra[r0 + r, sl] = one / (one + jnp.exp(z))
            return rcarry

        lax.fori_loop(0, _CHUNK // 8, _frow, 0)

    def _block(g, carry):
        pltpu.sync_copy(srcr.at[s, pl.ds(g * _IBLK, _IBLK)], sg)
        pltpu.sync_copy(dstr.at[s, pl.ds(g * _IBLK, _IBLK)], ds)
        for r in range(_IBLK):
            for j in range(_CHUNK // 16):
                sl = pl.ds(j * 16, 16)
                sg[r, sl] = sg[r, sl] + bias
                dg[r, sl] = jnp.minimum(ds[r, sl], _N - 1) + bias

        def _issue_g(i):
            rh, ra, gh, ga, _, _ = bufs[i % 3]
            dh = pltpu.async_copy(hs.at[sg.at[i]], rh, gh)
            da = pltpu.async_copy(av.at[dg.at[i]], ra, ga)
            return (dh, da)

        gd = [None] * _IBLK
        sd = [None] * _IBLK
        gd[0] = _issue_g(0)
        for i in range(_IBLK):
            rh, ra, _, _, sh, sf = bufs[i % 3]
            if i + 1 < _IBLK:
                if i - 2 >= 0:
                    sd[i - 2][0].wait()
                    sd[i - 2][1].wait()
                gd[i + 1] = _issue_g(i + 1)
            gd[i][0].wait()
            gd[i][1].wait()
            _compute(rh, ra)
            dh = pltpu.async_copy(rh, acc_h.at[ds.at[i]], sh, add=True)
            df = pltpu.async_copy(ra, acc_f.at[ds.at[i]], sf, add=True)
            sd[i] = (dh, df)
        for i in (_IBLK - 2, _IBLK - 1):
            sd[i][0].wait()
            sd[i][1].wait()
        return carry

    lax.fori_loop(0, _NBLK, _block, 0)
    plsc.subcore_barrier()

    obase = s * _OPT
    pltpu.sync_copy(acc_h.at[pl.ds(obase, _OPT)],
                    out_h.at[pl.ds(obase, _OPT), pl.ds(c * _HALF, _HALF)])
    pltpu.sync_copy(acc_f.at[pl.ds(obase, _OPT)],
                    out_f.at[pl.ds(obase, _OPT), pl.ds(c * _HALF, _HALF)])


_sc_seg = functools.partial(
    pl.kernel,
    out_type=[
        jax.ShapeDtypeStruct((_N, _DIM), jnp.float32),
        jax.ShapeDtypeStruct((_N, _DIM), jnp.float32),
    ],
    mesh=plsc.VectorSubcoreMesh(core_axis_name="c", subcore_axis_name="s"),
    scratch_types=[
        pltpu.VMEM((_IBLK, _CHUNK), jnp.int32),
        pltpu.VMEM((_IBLK, _CHUNK), jnp.int32),
        pltpu.VMEM((_IBLK, _CHUNK), jnp.int32),
        pltpu.VMEM((_CHUNK, _HALF), jnp.float32),
        pltpu.VMEM((_CHUNK, _HALF), jnp.float32),
        pltpu.VMEM((_CHUNK, _HALF), jnp.float32),
        pltpu.VMEM((_CHUNK, _HALF), jnp.float32),
        pltpu.VMEM((_CHUNK, _HALF), jnp.float32),
        pltpu.VMEM((_CHUNK, _HALF), jnp.float32),
        pltpu.VMEM((_HALF,), jnp.float32),
        pltpu.VMEM_SHARED((_NROWS, _HALF), jnp.float32),
        pltpu.VMEM_SHARED((_NROWS, _HALF), jnp.float32),
    ] + [pltpu.SemaphoreType.DMA] * 12,
    compiler_params=pltpu.CompilerParams(use_tc_tiling_on_sc=False),
)(_sc_body)


def _prep_body(x_ref, h_ref, w_ref, b_ref, hs_ref, as_ref):
    cidx = pl.program_id(0)
    xb = x_ref[...]
    hb = h_ref[...]
    ab = -(w_ref[...] * xb + b_ref[...])
    hs_ref[...] = jnp.where(cidx == 0, hb[:, 0, :], hb[:, 1, :])
    as_ref[...] = jnp.where(cidx == 0, ab[:, 0, :], ab[:, 1, :])


def _prep(x, h, w_for, b_for):
    blk = 1000
    full_in = pl.BlockSpec((blk, 2, _HALF), lambda c, i: (i, 0, 0))
    vec = pl.BlockSpec((1, 2, _HALF), lambda c, i: (0, 0, 0))
    out = pl.BlockSpec((blk, _HALF), lambda c, i: (c * (_N // blk) + i, 0))
    return pl.pallas_call(
        _prep_body,
        grid=(2, _N // blk),
        in_specs=[full_in, full_in, vec, vec],
        out_specs=[out, out],
        out_shape=[
            jax.ShapeDtypeStruct((2 * _N, _HALF), jnp.float32),
            jax.ShapeDtypeStruct((2 * _N, _HALF), jnp.float32),
        ],
    )(x.reshape(_N, 2, _HALF), h.reshape(_N, 2, _HALF),
      w_for.reshape(1, 2, _HALF), b_for.reshape(1, 2, _HALF))


def _gates_body(x_ref, hs_ref, fs_ref, wi, ui, bi, wc, uc, bc, wo, uo, bo,
                ht_ref, ct_ref):
    x = x_ref[...]
    hsum = hs_ref[...]
    fsum = fs_ref[...]
    it = jax.nn.sigmoid(wi[...] * x + ui[...] * hsum + bi[...])
    ctt = jnp.tanh(wc[...] * x + uc[...] * hsum + bc[...])
    ct = it * ctt + fsum
    ot = jax.nn.sigmoid(wo[...] * x + uo[...] * hsum + bo[...])
    ht_ref[...] = ot * jnp.tanh(ct)
    ct_ref[...] = ct


def _gates(x, hsum, fsum, wi, ui, bi, wc, uc, bc, wo, uo, bo):
    blk = 1000
    grid = _N // blk
    row = pl.BlockSpec((blk, _DIM), lambda i: (i, 0))
    vec = pl.BlockSpec((1, _DIM), lambda i: (0, 0))
    return pl.pallas_call(
        _gates_body,
        grid=(grid,),
        in_specs=[row, row, row] + [vec] * 9,
        out_specs=[row, row],
        out_shape=[
            jax.ShapeDtypeStruct((_N, _DIM), jnp.float32),
            jax.ShapeDtypeStruct((_N, _DIM), jnp.float32),
        ],
    )(x, hsum, fsum, wi, ui, bi, wc, uc, bc, wo, uo, bo)


def kernel(x, h, w_for, u_for, b_for, w_in, u_in, b_in, w_ce, u_ce, b_ce,
           w_out, u_out, b_out, edge_index):
    src = edge_index[0].astype(jnp.int32)
    dst = edge_index[1].astype(jnp.int32)
    pad = _EPAD - _E
    srcr = jnp.concatenate([src, jnp.zeros((pad,), jnp.int32)])
    dstr = jnp.concatenate([dst, jnp.full((pad,), _N, jnp.int32)])
    srcr = srcr.reshape(_NSUB, _CPT, _CHUNK)
    dstr = dstr.reshape(_NSUB, _CPT, _CHUNK)
    hs, av = _prep(x, h, w_for, b_for)

    chs, cfs = _sc_seg(hs, av, srcr, dstr, u_for)

    r = lambda v: v.reshape(1, _DIM)
    ht, ct = _gates(x, chs, cfs, r(w_in), r(u_in), r(b_in), r(w_ce), r(u_ce),
                    r(b_ce), r(w_out), r(u_out), r(b_out))
    return ht, ct
